# baseline (speedup 1.0000x reference)
import functools
import math

import jax
import jax.numpy as jnp
from jax import lax
from jax.experimental import pallas as pl
from jax.experimental.pallas import tpu as pltpu
from jax.experimental.pallas import tpu_sc as plsc

D_MODEL = 64
SCALE = math.sqrt(D_MODEL)
LANES = 16
SUBCHUNK = 128


@functools.lru_cache(maxsize=None)
def _build(nw: int, nc: int, nchunks: int):
    mesh = plsc.VectorSubcoreMesh(core_axis_name="c", subcore_axis_name="s")
    S = SUBCHUNK
    D = D_MODEL

    @functools.partial(
        pl.kernel,
        out_type=jax.ShapeDtypeStruct((nw, nchunks, S, D), jnp.float32),
        mesh=mesh,
        scratch_types=[
            pltpu.VMEM((nchunks, S), jnp.int32),
            pltpu.VMEM((S, D), jnp.float32),
            pltpu.VMEM((S, D), jnp.float32),
            pltpu.SemaphoreType.DMA,
        ],
        compiler_params=pltpu.CompilerParams(use_tc_tiling_on_sc=False),
    )
    def k(idx_hbm, table_hbm, out_hbm, idx_v, gbuf, obuf, gsem):
        wid = lax.axis_index("s") * nc + lax.axis_index("c")
        pltpu.sync_copy(idx_hbm.at[wid], idx_v)

        @pl.loop(0, nchunks)
        def _chunk(g):
            pltpu.async_copy(table_hbm.at[idx_v.at[g]], gbuf, gsem).wait()

            @pl.loop(0, S)
            def _row(i):
                for j in range(D // LANES):
                    sl = pl.ds(j * LANES, LANES)
                    obuf[i, sl] = gbuf[i, sl] * SCALE

            pltpu.sync_copy(obuf, out_hbm.at[wid, g])

    return k


def kernel(x, table):
    info = plsc.get_sparse_core_info()
    nc, ns = info.num_cores, info.num_subcores
    nw = nc * ns
    orig_shape = x.shape
    b = x.size
    xf = x.reshape(-1).astype(jnp.int32)
    block = nw * SUBCHUNK
    pad = (-b) % block
    if pad:
        xf = jnp.concatenate([xf, jnp.zeros((pad,), jnp.int32)])
    nchunks = (b + pad) // block
    xr = xf.reshape(nw, nchunks, SUBCHUNK)
    out = _build(nw, nc, nchunks)(xr, table)
    out = out.reshape(-1, D_MODEL)
    if pad:
        out = out[:b]
    return out.reshape(*orig_shape, D_MODEL)


# speedup vs baseline: 1.0791x; 1.0791x over previous
import functools
import math

import jax
import jax.numpy as jnp
from jax import lax
from jax.experimental import pallas as pl
from jax.experimental.pallas import tpu as pltpu
from jax.experimental.pallas import tpu_sc as plsc

D_MODEL = 64
SCALE = math.sqrt(D_MODEL)
LANES = 16
SUBCHUNK = 128


@functools.lru_cache(maxsize=None)
def _build(nw: int, nc: int, nchunks: int):
    mesh = plsc.VectorSubcoreMesh(core_axis_name="c", subcore_axis_name="s")
    S = SUBCHUNK
    D = D_MODEL
    nbuf = next(n for n in (4, 2, 1) if nchunks % n == 0)

    @functools.partial(
        pl.kernel,
        out_type=jax.ShapeDtypeStruct((nw, nchunks, S, D), jnp.float32),
        mesh=mesh,
        scratch_types=[
            pltpu.VMEM((nchunks, S), jnp.int32),
            pltpu.VMEM((nbuf, S, D), jnp.float32),
            pltpu.VMEM((nbuf, S, D), jnp.float32),
        ]
        + [pltpu.SemaphoreType.DMA] * (2 * nbuf),
        compiler_params=pltpu.CompilerParams(use_tc_tiling_on_sc=False),
    )
    def k(idx_hbm, table_hbm, out_hbm, idx_v, gbuf, obuf, *sems):
        gsem, ssem = sems[:nbuf], sems[nbuf:]
        wid = lax.axis_index("s") * nc + lax.axis_index("c")
        pltpu.sync_copy(idx_hbm.at[wid], idx_v)

        for b in range(nbuf):
            pltpu.async_copy(table_hbm.at[idx_v.at[b]], gbuf.at[b], gsem[b])

        @pl.loop(0, nchunks, step=nbuf)
        def _outer(g0):
            for b in range(nbuf):
                g = g0 + b
                pltpu.make_async_copy(
                    table_hbm.at[idx_v.at[g]], gbuf.at[b], gsem[b]
                ).wait()

                @pl.when(g0 > 0)
                def _():
                    pltpu.make_async_copy(
                        obuf.at[b], out_hbm.at[wid, g], ssem[b]
                    ).wait()

                @pl.loop(0, S, unroll=4)
                def _row(i):
                    for j in range(D // LANES):
                        sl = pl.ds(j * LANES, LANES)
                        obuf[b, i, sl] = gbuf[b, i, sl] * SCALE

                pltpu.async_copy(obuf.at[b], out_hbm.at[wid, g], ssem[b])

                @pl.when(g0 + nbuf < nchunks)
                def _():
                    pltpu.async_copy(
                        table_hbm.at[idx_v.at[g + nbuf]], gbuf.at[b], gsem[b]
                    )

        for b in range(nbuf):
            pltpu.make_async_copy(
                obuf.at[b], out_hbm.at[wid, nchunks - nbuf + b], ssem[b]
            ).wait()

    return k


def kernel(x, table):
    info = plsc.get_sparse_core_info()
    nc, ns = info.num_cores, info.num_subcores
    nw = nc * ns
    orig_shape = x.shape
    b = x.size
    xf = x.reshape(-1).astype(jnp.int32)
    block = nw * SUBCHUNK
    pad = (-b) % block
    if pad:
        xf = jnp.concatenate([xf, jnp.zeros((pad,), jnp.int32)])
    nchunks = (b + pad) // block
    xr = xf.reshape(nw, nchunks, SUBCHUNK)
    out = _build(nw, nc, nchunks)(xr, table)
    out = out.reshape(-1, D_MODEL)
    if pad:
        out = out[:b]
    return out.reshape(*orig_shape, D_MODEL)
